# baseline (speedup 1.0000x reference)
import jax
import jax.numpy as jnp
from jax import lax
from jax.experimental import pallas as pl
from jax.experimental.pallas import tpu as pltpu
from jax.experimental.pallas import tpu_sc as plsc

_N = 5000
_FLAT = _N * 2
_L = 16
_NW = 16
_PAD = 10240
_CHUNK = _PAD // _NW
_VECS = _CHUNK // _L
_HALF = _CHUNK // 2
_T = 1000
_TPAD = 1024


def _sc_body(t_hbm, ac_hbm, om_hbm, tb_hbm, nz_hbm, out_hbm,
             t_v, ac_v, om_v, tb_v, nz_v, o_v,
             sem_t, sem_tab, sem_in, sem_out):
    wid = lax.axis_index("s")
    base = wid * _CHUNK
    cp_t = pltpu.async_copy(t_hbm, t_v, sem_t)
    cp_ac = pltpu.async_copy(ac_hbm, ac_v, sem_tab)
    cp_om = pltpu.async_copy(om_hbm, om_v, sem_tab)
    cp_tb = pltpu.async_copy(tb_hbm.at[pl.ds(base, _CHUNK)], tb_v, sem_in)
    cp_nz = pltpu.async_copy(nz_hbm.at[pl.ds(base, _CHUNK)], nz_v, sem_in)
    cp_t.wait()
    cp_ac.wait()
    cp_om.wait()
    ts = t_v[...][0]
    c1 = jnp.full((_L,), ac_v[pl.ds(ts, _L)][0], jnp.float32)
    c2 = jnp.full((_L,), om_v[pl.ds(ts, _L)][0], jnp.float32)
    cp_tb.wait()
    cp_nz.wait()
    cp_lo = None
    for j in range(_VECS):
        sl = pl.ds(j * _L, _L)
        x = tb_v[sl] * 2.0 - 1.0
        y = c1 * x + c2 * nz_v[sl]
        y = jnp.clip(y, -1.0, 1.0)
        o_v[sl] = (y + 1.0) * 0.5
        if j == _VECS // 2 - 1:
            cp_lo = pltpu.async_copy(
                o_v.at[pl.ds(0, _HALF)],
                out_hbm.at[pl.ds(base, _HALF)], sem_out)
    cp_hi = pltpu.async_copy(
        o_v.at[pl.ds(_HALF, _HALF)],
        out_hbm.at[pl.ds(base + _HALF, _HALF)], sem_out)
    cp_lo.wait()
    cp_hi.wait()


@jax.jit
def kernel(audio, true_boxes, sqrt_alphas_cumprod, sqrt_one_minus_alphas_cumprod, noise, t):
    del audio
    tb = jnp.pad(true_boxes.reshape(_FLAT), (0, _PAD - _FLAT))
    nz = jnp.pad(noise.reshape(_FLAT), (0, _PAD - _FLAT))
    ac = jnp.pad(sqrt_alphas_cumprod, (0, _TPAD - _T))
    om = jnp.pad(sqrt_one_minus_alphas_cumprod, (0, _TPAD - _T))
    t16 = jnp.broadcast_to(t.astype(jnp.int32), (_L,))
    run = pl.kernel(
        _sc_body,
        out_type=jax.ShapeDtypeStruct((_PAD,), jnp.float32),
        mesh=plsc.VectorSubcoreMesh(core_axis_name="c", subcore_axis_name="s",
                                    num_cores=1),
        scratch_types=[
            pltpu.VMEM((_L,), jnp.int32),
            pltpu.VMEM((_TPAD,), jnp.float32),
            pltpu.VMEM((_TPAD,), jnp.float32),
            pltpu.VMEM((_CHUNK,), jnp.float32),
            pltpu.VMEM((_CHUNK,), jnp.float32),
            pltpu.VMEM((_CHUNK,), jnp.float32),
            pltpu.SemaphoreType.DMA,
            pltpu.SemaphoreType.DMA,
            pltpu.SemaphoreType.DMA,
            pltpu.SemaphoreType.DMA,
        ],
    )
    out = run(t16, ac, om, tb, nz)
    return out[:_FLAT].reshape(_N, 2)


# speedup vs baseline: 1.0094x; 1.0094x over previous
import jax
import jax.numpy as jnp
from jax import lax
from jax.experimental import pallas as pl
from jax.experimental.pallas import tpu as pltpu
from jax.experimental.pallas import tpu_sc as plsc

_N = 5000
_FLAT = _N * 2
_L = 16
_NW = 16
_PAD = 10240
_CHUNK = _PAD // _NW
_VECS = _CHUNK // _L
_T = 1000
_TPAD = 1024


def _sc_body(t_hbm, ac_hbm, om_hbm, tb_hbm, nz_hbm, out_hbm,
             t_v, ac_v, om_v, tb_v, nz_v, o_v,
             sem_t, sem_tab, sem_in, sem_out):
    wid = lax.axis_index("s")
    base = wid * _CHUNK
    cp_t = pltpu.async_copy(t_hbm, t_v, sem_t)
    cp_ac = pltpu.async_copy(ac_hbm, ac_v, sem_tab)
    cp_om = pltpu.async_copy(om_hbm, om_v, sem_tab)
    cp_tb = pltpu.async_copy(tb_hbm.at[pl.ds(base, _CHUNK)], tb_v, sem_in)
    cp_nz = pltpu.async_copy(nz_hbm.at[pl.ds(base, _CHUNK)], nz_v, sem_in)
    cp_t.wait()
    cp_ac.wait()
    cp_om.wait()
    ts = t_v[...][0]
    c1 = jnp.full((_L,), ac_v[pl.ds(ts, _L)][0], jnp.float32)
    c2 = jnp.full((_L,), om_v[pl.ds(ts, _L)][0], jnp.float32)
    cp_tb.wait()
    cp_nz.wait()

    def step(j, carry):
        sl = pl.ds(j * _L, _L)
        x = tb_v[sl] * 2.0 - 1.0
        y = c1 * x + c2 * nz_v[sl]
        y = jnp.clip(y, -1.0, 1.0)
        o_v[sl] = (y + 1.0) * 0.5
        return carry

    lax.fori_loop(0, _VECS, step, 0)
    pltpu.sync_copy(o_v, out_hbm.at[pl.ds(base, _CHUNK)])


@jax.jit
def kernel(audio, true_boxes, sqrt_alphas_cumprod, sqrt_one_minus_alphas_cumprod, noise, t):
    del audio
    tb = jnp.pad(true_boxes.reshape(_FLAT), (0, _PAD - _FLAT))
    nz = jnp.pad(noise.reshape(_FLAT), (0, _PAD - _FLAT))
    ac = jnp.pad(sqrt_alphas_cumprod, (0, _TPAD - _T))
    om = jnp.pad(sqrt_one_minus_alphas_cumprod, (0, _TPAD - _T))
    t16 = jnp.broadcast_to(t.astype(jnp.int32), (_L,))
    run = pl.kernel(
        _sc_body,
        out_type=jax.ShapeDtypeStruct((_PAD,), jnp.float32),
        mesh=plsc.VectorSubcoreMesh(core_axis_name="c", subcore_axis_name="s",
                                    num_cores=1),
        scratch_types=[
            pltpu.VMEM((_L,), jnp.int32),
            pltpu.VMEM((_TPAD,), jnp.float32),
            pltpu.VMEM((_TPAD,), jnp.float32),
            pltpu.VMEM((_CHUNK,), jnp.float32),
            pltpu.VMEM((_CHUNK,), jnp.float32),
            pltpu.VMEM((_CHUNK,), jnp.float32),
            pltpu.SemaphoreType.DMA,
            pltpu.SemaphoreType.DMA,
            pltpu.SemaphoreType.DMA,
            pltpu.SemaphoreType.DMA,
        ],
    )
    out = run(t16, ac, om, tb, nz)
    return out[:_FLAT].reshape(_N, 2)


# speedup vs baseline: 1.0938x; 1.0836x over previous
import jax
import jax.numpy as jnp
from jax import lax
from jax.experimental import pallas as pl
from jax.experimental.pallas import tpu as pltpu
from jax.experimental.pallas import tpu_sc as plsc

_N = 5000
_FLAT = _N * 2
_L = 16
_NW = 16
_CHUNK = 640
_VECS = _CHUNK // _L
_LASTBASE = _FLAT - _CHUNK
_T = 1000
_TPAD = 1024


def _sc_body(t_hbm, ac_hbm, om_hbm, tb_hbm, nz_hbm, out_hbm,
             t_v, ac_v, om_v, tb_v, nz_v, o_v,
             sem_t, sem_tab, sem_in, sem_out):
    wid = lax.axis_index("s")
    base = jnp.minimum(wid * _CHUNK, _LASTBASE)
    cp_t = pltpu.async_copy(t_hbm, t_v.at[pl.ds(0, 1)], sem_t)
    cp_ac = pltpu.async_copy(ac_hbm, ac_v.at[pl.ds(0, _T)], sem_tab)
    cp_om = pltpu.async_copy(om_hbm, om_v.at[pl.ds(0, _T)], sem_tab)
    cp_tb = pltpu.async_copy(tb_hbm.at[pl.ds(base, _CHUNK)], tb_v, sem_in)
    cp_nz = pltpu.async_copy(nz_hbm.at[pl.ds(base, _CHUNK)], nz_v, sem_in)
    cp_t.wait()
    cp_ac.wait()
    cp_om.wait()
    ts = t_v[...][0]
    c1 = jnp.full((_L,), ac_v[pl.ds(ts, _L)][0], jnp.float32)
    c2 = jnp.full((_L,), om_v[pl.ds(ts, _L)][0], jnp.float32)
    cp_tb.wait()
    cp_nz.wait()

    def step(j, carry):
        sl = pl.ds(j * _L, _L)
        x = tb_v[sl] * 2.0 - 1.0
        y = c1 * x + c2 * nz_v[sl]
        y = jnp.clip(y, -1.0, 1.0)
        o_v[sl] = (y + 1.0) * 0.5
        return carry

    lax.fori_loop(0, _VECS, step, 0)
    pltpu.sync_copy(o_v, out_hbm.at[pl.ds(base, _CHUNK)])


@jax.jit
def kernel(audio, true_boxes, sqrt_alphas_cumprod, sqrt_one_minus_alphas_cumprod, noise, t):
    del audio
    run = pl.kernel(
        _sc_body,
        out_type=jax.ShapeDtypeStruct((_FLAT,), jnp.float32),
        mesh=plsc.VectorSubcoreMesh(core_axis_name="c", subcore_axis_name="s",
                                    num_cores=1),
        scratch_types=[
            pltpu.VMEM((_L,), jnp.int32),
            pltpu.VMEM((_TPAD,), jnp.float32),
            pltpu.VMEM((_TPAD,), jnp.float32),
            pltpu.VMEM((_CHUNK,), jnp.float32),
            pltpu.VMEM((_CHUNK,), jnp.float32),
            pltpu.VMEM((_CHUNK,), jnp.float32),
            pltpu.SemaphoreType.DMA,
            pltpu.SemaphoreType.DMA,
            pltpu.SemaphoreType.DMA,
            pltpu.SemaphoreType.DMA,
        ],
    )
    out = run(t.astype(jnp.int32), sqrt_alphas_cumprod,
              sqrt_one_minus_alphas_cumprod,
              true_boxes.reshape(_FLAT), noise.reshape(_FLAT))
    return out.reshape(_N, 2)
